# baseline (speedup 1.0000x reference)
import functools

import jax
import jax.numpy as jnp
from jax import lax
from jax.experimental import pallas as pl
from jax.experimental.pallas import tpu as pltpu
from jax.experimental.pallas import tpu_sc as plsc

K = 128


def _enc_kernel(x_ref, w_ref, b_ref, pre_ref):
    acc = jax.lax.dot_general(
        x_ref[...], w_ref[...],
        dimension_numbers=(((1,), (0,)), ((), ())),
        preferred_element_type=jnp.float32,
    )
    pre_ref[...] = jnp.maximum(acc + b_ref[...], 0.0)


def _encode(x, w, b, *, block_m=512, block_n=1024):
    m, d = x.shape
    n = w.shape[1]
    grid = (n // block_n, m // block_m)
    return pl.pallas_call(
        _enc_kernel,
        grid=grid,
        in_specs=[
            pl.BlockSpec((block_m, d), lambda j, i: (i, 0)),
            pl.BlockSpec((d, block_n), lambda j, i: (0, j)),
            pl.BlockSpec((block_n,), lambda j, i: (j,)),
        ],
        out_specs=pl.BlockSpec((block_m, block_n), lambda j, i: (i, j)),
        out_shape=jax.ShapeDtypeStruct((m, n), jnp.float32),
    )(x, w, b)


def _topk_mask_kernel(pre_ref, z_ref, zb_ref, q_ref):
    y = pre_ref[...]
    yi = jax.lax.bitcast_convert_type(y, jnp.int32)
    bm = y.shape[0]
    q_ref[...] = (yi >> 16).astype(jnp.int16)

    def s1(it, t):
        cand = t | (1 << (14 - it))
        ch = cand.astype(jnp.int16)
        h = q_ref[...]
        ind = (h >= ch).astype(jnp.int16)
        cnt = jnp.sum(ind, axis=1, keepdims=True, dtype=jnp.int32)
        return jnp.where(cnt >= K, cand, t)

    t16 = jax.lax.fori_loop(0, 15, s1, jnp.zeros((bm, 1), jnp.int32))

    h = q_ref[...]
    cnt_gt = jnp.sum((h > t16.astype(jnp.int16)).astype(jnp.int16),
                     axis=1, keepdims=True, dtype=jnp.int32)
    r = K - cnt_gt

    lo = ((yi & 0xFFFF) - 32768).astype(jnp.int16)
    q_ref[...] = jnp.where(h == t16.astype(jnp.int16), lo, jnp.int16(-32768))

    def s2(it, t):
        cand = t | (1 << (15 - it))
        cs = (cand - 32768).astype(jnp.int16)
        ind = (q_ref[...] >= cs).astype(jnp.int16)
        cnt = jnp.sum(ind, axis=1, keepdims=True, dtype=jnp.int32)
        return jnp.where(cnt >= r, cand, t)

    tlo = jax.lax.fori_loop(0, 16, s2, jnp.zeros((bm, 1), jnp.int32))

    vk = jax.lax.bitcast_convert_type((t16 << 16) | tlo, jnp.float32)
    z = jnp.where(y >= vk, y, 0.0)
    z_ref[...] = z
    zb_ref[...] = z.astype(jnp.bfloat16)


def _topk_mask(pre, *, block_m=64):
    m, n = pre.shape
    return pl.pallas_call(
        _topk_mask_kernel,
        grid=(m // block_m,),
        in_specs=[pl.BlockSpec((block_m, n), lambda i: (i, 0))],
        out_specs=[
            pl.BlockSpec((block_m, n), lambda i: (i, 0)),
            pl.BlockSpec((block_m, n), lambda i: (i, 0)),
        ],
        out_shape=[
            jax.ShapeDtypeStruct((m, n), jnp.float32),
            jax.ShapeDtypeStruct((m, n), jnp.bfloat16),
        ],
        scratch_shapes=[pltpu.VMEM((block_m, n), jnp.int16)],
    )(pre)


def _sc_row_threshold(pre):
    m, n = pre.shape
    nchunk = 128
    chunk = n // nchunk
    nvec = chunk // 16
    mesh = plsc.VectorSubcoreMesh(core_axis_name="c", subcore_axis_name="s")
    nworker = 32
    rows_per = m // nworker

    @functools.partial(
        pl.kernel,
        out_type=jax.ShapeDtypeStruct((m,), jnp.float32),
        mesh=mesh,
        scratch_types=[
            pltpu.VMEM((n,), jnp.float32),
            pltpu.VMEM((n + 16,), jnp.float32),
            pltpu.VMEM((rows_per,), jnp.float32),
            pltpu.VMEM((16,), jnp.float32),
        ],
    )
    def body(pre_hbm, out_hbm, row_v, comp_v, thr_v, tmp_v):
        wid = lax.axis_index("s") * 2 + lax.axis_index("c")
        base = wid * rows_per
        lane0 = lax.iota(jnp.int32, 16) == 0

        def per_row(i, _):
            pltpu.sync_copy(pre_hbm.at[base + i], row_v)

            def group_max(c, ms):
                return tuple(
                    jnp.maximum(ms[g], row_v[pl.ds((c * 8 + g) * 16, 16)])
                    for g in range(8)
                )

            zeros = jnp.zeros((16,), jnp.float32)
            ms = lax.fori_loop(0, n // 128, group_max, (zeros,) * 8)
            lane_min = ms[0]
            for g in range(1, 8):
                lane_min = jnp.minimum(lane_min, ms[g])
            t0 = lane_min[0]
            for k in range(1, 16):
                t0 = jnp.minimum(t0, lane_min[k])
            t0v = jnp.full((16,), t0, jnp.float32)

            def compress(j, off):
                v = row_v[pl.ds(j * 16, 16)]
                msk = v >= t0v
                plsc.store_compressed(comp_v.at[pl.ds(off, 16)], v, mask=msk)
                return off + plsc.all_reduce_population_count(msk)[0]

            cnt = lax.fori_loop(0, n // 16, compress, jnp.int32(0))
            comp_v[pl.ds(cnt, 16)] = jnp.zeros((16,), jnp.float32)
            nv = (cnt + 15) // 16

            kv = jnp.full((16,), K, jnp.int32)

            def bit_step(b, t):
                candv = t | (jnp.full((16,), 1, jnp.int32) << (30 - b))

                def count(j, acc):
                    v = plsc.bitcast(comp_v[pl.ds(j * 16, 16)], jnp.int32)
                    return acc + plsc.all_reduce_population_count(v >= candv)

                cnt_ge = lax.fori_loop(0, nv, count, jnp.zeros((16,), jnp.int32))
                return jnp.where(cnt_ge >= kv, candv, t)

            t = lax.fori_loop(0, 31, bit_step, jnp.zeros((16,), jnp.int32))
            tf = plsc.bitcast(t, jnp.float32)
            plsc.store_scatter(thr_v, [jnp.full((16,), i, jnp.int32)], tf, mask=lane0)
            return 0

        lax.fori_loop(0, rows_per, per_row, 0)
        pltpu.sync_copy(thr_v, out_hbm.at[pl.ds(base, rows_per)])

    return body(pre)


def _mask_kernel(pre_ref, thr_ref, z_ref, zb_ref):
    y = pre_ref[...]
    z = jnp.where(y >= thr_ref[...], y, 0.0)
    z_ref[...] = z
    zb_ref[...] = z.astype(jnp.bfloat16)


def _mask(pre, thr, *, block_m=128):
    m, n = pre.shape
    return pl.pallas_call(
        _mask_kernel,
        grid=(m // block_m,),
        in_specs=[
            pl.BlockSpec((block_m, n), lambda i: (i, 0)),
            pl.BlockSpec((block_m, 1), lambda i: (i, 0)),
        ],
        out_specs=[
            pl.BlockSpec((block_m, n), lambda i: (i, 0)),
            pl.BlockSpec((block_m, n), lambda i: (i, 0)),
        ],
        out_shape=[
            jax.ShapeDtypeStruct((m, n), jnp.float32),
            jax.ShapeDtypeStruct((m, n), jnp.bfloat16),
        ],
    )(pre, thr)


def _dec_kernel(z_ref, w_ref, b_ref, out_ref):
    acc = jax.lax.dot_general(
        z_ref[...], w_ref[...],
        dimension_numbers=(((1,), (0,)), ((), ())),
        preferred_element_type=jnp.float32,
    )
    out_ref[...] = acc + b_ref[...]


def _decode(zb, w, b, *, block_m=256, block_n=512):
    m, h = zb.shape
    n = w.shape[1]
    grid = (n // block_n, m // block_m)
    return pl.pallas_call(
        _dec_kernel,
        grid=grid,
        in_specs=[
            pl.BlockSpec((block_m, h), lambda j, i: (i, 0)),
            pl.BlockSpec((h, block_n), lambda j, i: (0, j)),
            pl.BlockSpec((block_n,), lambda j, i: (j,)),
        ],
        out_specs=pl.BlockSpec((block_m, block_n), lambda j, i: (i, j)),
        out_shape=jax.ShapeDtypeStruct((m, n), jnp.float32),
    )(zb, w, b)


def kernel(x, W_enc, b_enc, W_dec, b_dec):
    pre = _encode(x.astype(jnp.bfloat16), W_enc.astype(jnp.bfloat16), b_enc)
    z, zb = _topk_mask(pre)
    x_recon = _decode(zb, W_dec.astype(jnp.bfloat16), b_dec)
    return (z, x_recon)


# speedup vs baseline: 1.3398x; 1.3398x over previous
import functools

import jax
import jax.numpy as jnp
from jax import lax
from jax.experimental import pallas as pl
from jax.experimental.pallas import tpu as pltpu
from jax.experimental.pallas import tpu_sc as plsc

K = 128


def _enc_kernel(x_ref, w_ref, b_ref, pre_ref):
    acc = jax.lax.dot_general(
        x_ref[...], w_ref[...],
        dimension_numbers=(((1,), (0,)), ((), ())),
        preferred_element_type=jnp.float32,
    )
    pre_ref[...] = jnp.maximum(acc + b_ref[...], 0.0)


def _encode(x, w, b, *, block_m=512, block_n=2048):
    m, d = x.shape
    n = w.shape[1]
    grid = (n // block_n, m // block_m)
    return pl.pallas_call(
        _enc_kernel,
        grid=grid,
        in_specs=[
            pl.BlockSpec((block_m, d), lambda j, i: (i, 0)),
            pl.BlockSpec((d, block_n), lambda j, i: (0, j)),
            pl.BlockSpec((block_n,), lambda j, i: (j,)),
        ],
        out_specs=pl.BlockSpec((block_m, block_n), lambda j, i: (i, j)),
        out_shape=jax.ShapeDtypeStruct((m, n), jnp.float32),
    )(x, w, b)


def _topk_mask_kernel(pre_ref, z_ref, zb_ref):
    y = pre_ref[...]
    yi = jax.lax.bitcast_convert_type(y, jnp.int32)

    def body(it, t):
        cand = t | (1 << (30 - it))
        cnt = jnp.sum((yi >= cand).astype(jnp.int32), axis=1, keepdims=True)
        return jnp.where(cnt >= K, cand, t)

    t = jax.lax.fori_loop(0, 31, body, jnp.zeros((y.shape[0], 1), jnp.int32))
    z = jnp.where(yi >= t, y, 0.0)
    z_ref[...] = z
    zb_ref[...] = z.astype(jnp.bfloat16)


def _topk_mask(pre, *, block_m=128):
    m, n = pre.shape
    return pl.pallas_call(
        _topk_mask_kernel,
        grid=(m // block_m,),
        in_specs=[pl.BlockSpec((block_m, n), lambda i: (i, 0))],
        out_specs=[
            pl.BlockSpec((block_m, n), lambda i: (i, 0)),
            pl.BlockSpec((block_m, n), lambda i: (i, 0)),
        ],
        out_shape=[
            jax.ShapeDtypeStruct((m, n), jnp.float32),
            jax.ShapeDtypeStruct((m, n), jnp.bfloat16),
        ],
    )(pre)


def _sc_row_threshold(pre):
    m, n = pre.shape
    nchunk = 128
    chunk = n // nchunk
    nvec = chunk // 16
    mesh = plsc.VectorSubcoreMesh(core_axis_name="c", subcore_axis_name="s")
    nworker = 32
    rows_per = m // nworker

    @functools.partial(
        pl.kernel,
        out_type=jax.ShapeDtypeStruct((m,), jnp.float32),
        mesh=mesh,
        scratch_types=[
            pltpu.VMEM((n,), jnp.float32),
            pltpu.VMEM((n + 16,), jnp.float32),
            pltpu.VMEM((rows_per,), jnp.float32),
            pltpu.VMEM((16,), jnp.float32),
        ],
    )
    def body(pre_hbm, out_hbm, row_v, comp_v, thr_v, tmp_v):
        wid = lax.axis_index("s") * 2 + lax.axis_index("c")
        base = wid * rows_per
        lane0 = lax.iota(jnp.int32, 16) == 0

        def per_row(i, _):
            pltpu.sync_copy(pre_hbm.at[base + i], row_v)

            def group_max(c, ms):
                return tuple(
                    jnp.maximum(ms[g], row_v[pl.ds((c * 8 + g) * 16, 16)])
                    for g in range(8)
                )

            zeros = jnp.zeros((16,), jnp.float32)
            ms = lax.fori_loop(0, n // 128, group_max, (zeros,) * 8)
            lane_min = ms[0]
            for g in range(1, 8):
                lane_min = jnp.minimum(lane_min, ms[g])
            t0 = lane_min[0]
            for k in range(1, 16):
                t0 = jnp.minimum(t0, lane_min[k])
            t0v = jnp.full((16,), t0, jnp.float32)

            def compress(j, off):
                v = row_v[pl.ds(j * 16, 16)]
                msk = v >= t0v
                plsc.store_compressed(comp_v.at[pl.ds(off, 16)], v, mask=msk)
                return off + plsc.all_reduce_population_count(msk)[0]

            cnt = lax.fori_loop(0, n // 16, compress, jnp.int32(0))
            comp_v[pl.ds(cnt, 16)] = jnp.zeros((16,), jnp.float32)
            nv = (cnt + 15) // 16

            kv = jnp.full((16,), K, jnp.int32)

            def bit_step(b, t):
                candv = t | (jnp.full((16,), 1, jnp.int32) << (30 - b))

                def count(j, acc):
                    v = plsc.bitcast(comp_v[pl.ds(j * 16, 16)], jnp.int32)
                    return acc + plsc.all_reduce_population_count(v >= candv)

                cnt_ge = lax.fori_loop(0, nv, count, jnp.zeros((16,), jnp.int32))
                return jnp.where(cnt_ge >= kv, candv, t)

            t = lax.fori_loop(0, 31, bit_step, jnp.zeros((16,), jnp.int32))
            tf = plsc.bitcast(t, jnp.float32)
            plsc.store_scatter(thr_v, [jnp.full((16,), i, jnp.int32)], tf, mask=lane0)
            return 0

        lax.fori_loop(0, rows_per, per_row, 0)
        pltpu.sync_copy(thr_v, out_hbm.at[pl.ds(base, rows_per)])

    return body(pre)


def _mask_kernel(pre_ref, thr_ref, z_ref, zb_ref):
    y = pre_ref[...]
    z = jnp.where(y >= thr_ref[...], y, 0.0)
    z_ref[...] = z
    zb_ref[...] = z.astype(jnp.bfloat16)


def _mask(pre, thr, *, block_m=128):
    m, n = pre.shape
    return pl.pallas_call(
        _mask_kernel,
        grid=(m // block_m,),
        in_specs=[
            pl.BlockSpec((block_m, n), lambda i: (i, 0)),
            pl.BlockSpec((block_m, 1), lambda i: (i, 0)),
        ],
        out_specs=[
            pl.BlockSpec((block_m, n), lambda i: (i, 0)),
            pl.BlockSpec((block_m, n), lambda i: (i, 0)),
        ],
        out_shape=[
            jax.ShapeDtypeStruct((m, n), jnp.float32),
            jax.ShapeDtypeStruct((m, n), jnp.bfloat16),
        ],
    )(pre, thr)


def _dec_kernel(z_ref, w_ref, b_ref, out_ref):
    acc = jax.lax.dot_general(
        z_ref[...], w_ref[...],
        dimension_numbers=(((1,), (0,)), ((), ())),
        preferred_element_type=jnp.float32,
    )
    out_ref[...] = acc + b_ref[...]


def _decode(zb, w, b, *, block_m=256, block_n=512):
    m, h = zb.shape
    n = w.shape[1]
    grid = (n // block_n, m // block_m)
    return pl.pallas_call(
        _dec_kernel,
        grid=grid,
        in_specs=[
            pl.BlockSpec((block_m, h), lambda j, i: (i, 0)),
            pl.BlockSpec((h, block_n), lambda j, i: (0, j)),
            pl.BlockSpec((block_n,), lambda j, i: (j,)),
        ],
        out_specs=pl.BlockSpec((block_m, block_n), lambda j, i: (i, j)),
        out_shape=jax.ShapeDtypeStruct((m, n), jnp.float32),
    )(zb, w, b)


def kernel(x, W_enc, b_enc, W_dec, b_dec):
    pre = _encode(x.astype(jnp.bfloat16), W_enc.astype(jnp.bfloat16), b_enc)
    z, zb = _topk_mask(pre)
    x_recon = _decode(zb, W_dec.astype(jnp.bfloat16), b_dec)
    return (z, x_recon)


# speedup vs baseline: 1.3423x; 1.0019x over previous
import jax
import jax.numpy as jnp
from jax.experimental import pallas as pl

K = 128


def _enc_kernel(x_ref, w_ref, b_ref, pre_ref):
    acc = jax.lax.dot_general(
        x_ref[...], w_ref[...],
        dimension_numbers=(((1,), (0,)), ((), ())),
        preferred_element_type=jnp.float32,
    )
    pre_ref[...] = jnp.maximum(acc + b_ref[...], 0.0)


def _encode(x, w, b, *, block_m=512, block_n=2048):
    m, d = x.shape
    n = w.shape[1]
    grid = (n // block_n, m // block_m)
    return pl.pallas_call(
        _enc_kernel,
        grid=grid,
        in_specs=[
            pl.BlockSpec((block_m, d), lambda j, i: (i, 0)),
            pl.BlockSpec((d, block_n), lambda j, i: (0, j)),
            pl.BlockSpec((block_n,), lambda j, i: (j,)),
        ],
        out_specs=pl.BlockSpec((block_m, block_n), lambda j, i: (i, j)),
        out_shape=jax.ShapeDtypeStruct((m, n), jnp.float32),
    )(x, w, b)


def _topk_mask_kernel(pre_ref, z_ref, zb_ref):
    y = pre_ref[...]
    yi = jax.lax.bitcast_convert_type(y, jnp.int32)

    def body(it, t):
        cand = t | (1 << (30 - it))
        cnt = jnp.sum((yi >= cand).astype(jnp.int32), axis=1, keepdims=True)
        return jnp.where(cnt >= K, cand, t)

    t = jax.lax.fori_loop(0, 31, body, jnp.zeros((y.shape[0], 1), jnp.int32))
    z = jnp.where(yi >= t, y, 0.0)
    z_ref[...] = z
    zb_ref[...] = z.astype(jnp.bfloat16)


def _topk_mask(pre, *, block_m=128):
    m, n = pre.shape
    return pl.pallas_call(
        _topk_mask_kernel,
        grid=(m // block_m,),
        in_specs=[pl.BlockSpec((block_m, n), lambda i: (i, 0))],
        out_specs=[
            pl.BlockSpec((block_m, n), lambda i: (i, 0)),
            pl.BlockSpec((block_m, n), lambda i: (i, 0)),
        ],
        out_shape=[
            jax.ShapeDtypeStruct((m, n), jnp.float32),
            jax.ShapeDtypeStruct((m, n), jnp.bfloat16),
        ],
    )(pre)


def _dec_kernel(z_ref, w_ref, b_ref, out_ref):
    acc = jax.lax.dot_general(
        z_ref[...], w_ref[...],
        dimension_numbers=(((1,), (0,)), ((), ())),
        preferred_element_type=jnp.float32,
    )
    out_ref[...] = acc + b_ref[...]


def _decode(zb, w, b, *, block_m=256, block_n=512):
    m, h = zb.shape
    n = w.shape[1]
    grid = (n // block_n, m // block_m)
    return pl.pallas_call(
        _dec_kernel,
        grid=grid,
        in_specs=[
            pl.BlockSpec((block_m, h), lambda j, i: (i, 0)),
            pl.BlockSpec((h, block_n), lambda j, i: (0, j)),
            pl.BlockSpec((block_n,), lambda j, i: (j,)),
        ],
        out_specs=pl.BlockSpec((block_m, block_n), lambda j, i: (i, j)),
        out_shape=jax.ShapeDtypeStruct((m, n), jnp.float32),
    )(zb, w, b)


def kernel(x, W_enc, b_enc, W_dec, b_dec):
    pre = _encode(x.astype(jnp.bfloat16), W_enc.astype(jnp.bfloat16), b_enc)
    z, zb = _topk_mask(pre)
    x_recon = _decode(zb, W_dec.astype(jnp.bfloat16), b_dec)
    return (z, x_recon)


# speedup vs baseline: 1.3549x; 1.0094x over previous
import jax
import jax.numpy as jnp
from jax.experimental import pallas as pl

K = 128


def _enc_kernel(x_ref, w_ref, b_ref, pre_ref):
    acc = jax.lax.dot_general(
        x_ref[...], w_ref[...],
        dimension_numbers=(((1,), (0,)), ((), ())),
        preferred_element_type=jnp.float32,
    )
    pre_ref[...] = jnp.maximum(acc + b_ref[...], 0.0)


def _encode(x, w, b, *, block_m=512, block_n=2048):
    m, d = x.shape
    n = w.shape[1]
    grid = (n // block_n, m // block_m)
    return pl.pallas_call(
        _enc_kernel,
        grid=grid,
        in_specs=[
            pl.BlockSpec((block_m, d), lambda j, i: (i, 0)),
            pl.BlockSpec((d, block_n), lambda j, i: (0, j)),
            pl.BlockSpec((block_n,), lambda j, i: (j,)),
        ],
        out_specs=pl.BlockSpec((block_m, block_n), lambda j, i: (i, j)),
        out_shape=jax.ShapeDtypeStruct((m, n), jnp.float32),
    )(x, w, b)


def _topk_mask_kernel(pre_ref, z_ref, zb_ref):
    y = pre_ref[...]
    yi = jax.lax.bitcast_convert_type(y, jnp.int32)

    def body(it, t):
        cand = t | (1 << (30 - it))
        cnt = jnp.sum((yi >= cand).astype(jnp.int32), axis=1, keepdims=True)
        return jnp.where(cnt >= K, cand, t)

    t = jax.lax.fori_loop(0, 31, body, jnp.zeros((y.shape[0], 1), jnp.int32),
                          unroll=4)
    z = jnp.where(yi >= t, y, 0.0)
    z_ref[...] = z
    zb_ref[...] = z.astype(jnp.bfloat16)


def _topk_mask(pre, *, block_m=128):
    m, n = pre.shape
    return pl.pallas_call(
        _topk_mask_kernel,
        grid=(m // block_m,),
        in_specs=[pl.BlockSpec((block_m, n), lambda i: (i, 0))],
        out_specs=[
            pl.BlockSpec((block_m, n), lambda i: (i, 0)),
            pl.BlockSpec((block_m, n), lambda i: (i, 0)),
        ],
        out_shape=[
            jax.ShapeDtypeStruct((m, n), jnp.float32),
            jax.ShapeDtypeStruct((m, n), jnp.bfloat16),
        ],
    )(pre)


def _dec_kernel(z_ref, w_ref, b_ref, out_ref):
    acc = jax.lax.dot_general(
        z_ref[...], w_ref[...],
        dimension_numbers=(((1,), (0,)), ((), ())),
        preferred_element_type=jnp.float32,
    )
    out_ref[...] = acc + b_ref[...]


def _decode(zb, w, b, *, block_m=256, block_n=512):
    m, h = zb.shape
    n = w.shape[1]
    grid = (n // block_n, m // block_m)
    return pl.pallas_call(
        _dec_kernel,
        grid=grid,
        in_specs=[
            pl.BlockSpec((block_m, h), lambda j, i: (i, 0)),
            pl.BlockSpec((h, block_n), lambda j, i: (0, j)),
            pl.BlockSpec((block_n,), lambda j, i: (j,)),
        ],
        out_specs=pl.BlockSpec((block_m, block_n), lambda j, i: (i, j)),
        out_shape=jax.ShapeDtypeStruct((m, n), jnp.float32),
    )(zb, w, b)


def kernel(x, W_enc, b_enc, W_dec, b_dec):
    pre = _encode(x.astype(jnp.bfloat16), W_enc.astype(jnp.bfloat16), b_enc)
    z, zb = _topk_mask(pre)
    x_recon = _decode(zb, W_dec.astype(jnp.bfloat16), b_dec)
    return (z, x_recon)


# speedup vs baseline: 1.3577x; 1.0021x over previous
import jax
import jax.numpy as jnp
from jax.experimental import pallas as pl

K = 128


def _enc_kernel(x_ref, w_ref, b_ref, pre_ref):
    acc = jax.lax.dot_general(
        x_ref[...], w_ref[...],
        dimension_numbers=(((1,), (0,)), ((), ())),
        preferred_element_type=jnp.float32,
    )
    pre_ref[...] = jnp.maximum(acc + b_ref[...], 0.0)


def _encode(x, w, b, *, block_m=512, block_n=2048):
    m, d = x.shape
    n = w.shape[1]
    grid = (n // block_n, m // block_m)
    return pl.pallas_call(
        _enc_kernel,
        grid=grid,
        in_specs=[
            pl.BlockSpec((block_m, d), lambda j, i: (i, 0)),
            pl.BlockSpec((d, block_n), lambda j, i: (0, j)),
            pl.BlockSpec((block_n,), lambda j, i: (j,)),
        ],
        out_specs=pl.BlockSpec((block_m, block_n), lambda j, i: (i, j)),
        out_shape=jax.ShapeDtypeStruct((m, n), jnp.float32),
    )(x, w, b)


def _topk_mask_kernel(pre_ref, z_ref, zb_ref):
    y = pre_ref[...]
    yi = jax.lax.bitcast_convert_type(y, jnp.int32)

    def body(it, t):
        cand = t | (1 << (30 - it))
        cnt = jnp.sum((yi >= cand).astype(jnp.int32), axis=1, keepdims=True)
        return jnp.where(cnt >= K, cand, t)

    t = jax.lax.fori_loop(0, 31, body, jnp.zeros((y.shape[0], 1), jnp.int32),
                          unroll=8)
    z = jnp.where(yi >= t, y, 0.0)
    z_ref[...] = z
    zb_ref[...] = z.astype(jnp.bfloat16)


def _topk_mask(pre, *, block_m=128):
    m, n = pre.shape
    return pl.pallas_call(
        _topk_mask_kernel,
        grid=(m // block_m,),
        in_specs=[pl.BlockSpec((block_m, n), lambda i: (i, 0))],
        out_specs=[
            pl.BlockSpec((block_m, n), lambda i: (i, 0)),
            pl.BlockSpec((block_m, n), lambda i: (i, 0)),
        ],
        out_shape=[
            jax.ShapeDtypeStruct((m, n), jnp.float32),
            jax.ShapeDtypeStruct((m, n), jnp.bfloat16),
        ],
    )(pre)


def _dec_kernel(z_ref, w_ref, b_ref, out_ref):
    acc = jax.lax.dot_general(
        z_ref[...], w_ref[...],
        dimension_numbers=(((1,), (0,)), ((), ())),
        preferred_element_type=jnp.float32,
    )
    out_ref[...] = acc + b_ref[...]


def _decode(zb, w, b, *, block_m=256, block_n=512):
    m, h = zb.shape
    n = w.shape[1]
    grid = (n // block_n, m // block_m)
    return pl.pallas_call(
        _dec_kernel,
        grid=grid,
        in_specs=[
            pl.BlockSpec((block_m, h), lambda j, i: (i, 0)),
            pl.BlockSpec((h, block_n), lambda j, i: (0, j)),
            pl.BlockSpec((block_n,), lambda j, i: (j,)),
        ],
        out_specs=pl.BlockSpec((block_m, block_n), lambda j, i: (i, j)),
        out_shape=jax.ShapeDtypeStruct((m, n), jnp.float32),
    )(zb, w, b)


def kernel(x, W_enc, b_enc, W_dec, b_dec):
    pre = _encode(x.astype(jnp.bfloat16), W_enc.astype(jnp.bfloat16), b_enc)
    z, zb = _topk_mask(pre)
    x_recon = _decode(zb, W_dec.astype(jnp.bfloat16), b_dec)
    return (z, x_recon)
